# Initial kernel scaffold; baseline (speedup 1.0000x reference)
import jax
import jax.numpy as jnp
from jax.experimental import pallas as pl


def kernel(points, W1, g1, b1, W2, g2, b2, batch_size):
    raise NotImplementedError("write your pallas kernel here")


# speedup vs baseline: 2.1749x; 2.1749x over previous
import functools
import jax
import jax.numpy as jnp
from jax.experimental import pallas as pl
from jax.experimental.pallas import tpu as pltpu

VX, VY, VZ = 0.2, 0.2, 4.0
PCR = (0.0, -40.0, -3.0, 70.4, 40.0, 1.0)
CZ, CY, CX = 1, 400, 352
B = 2
N = 262144
C1, C2 = 64, 128
L = B * CZ * CY * CX
X_OFF = VX / 2 + PCR[0]
Y_OFF = VY / 2 + PCR[1]
Z_OFF = VZ / 2 + PCR[2]

BLK = 2048
GRID = N // BLK


def _voxelize_body(pts_ref, flat_ref, vals_ref):
    pts = pts_ref[...]
    bidx = pts[:, 0:1].astype(jnp.int32)
    x = pts[:, 1:2]
    y = pts[:, 2:3]
    z = pts[:, 3:4]
    cxf = jnp.clip(jnp.floor((x - PCR[0]) / VX), 0.0, CX - 1)
    cyf = jnp.clip(jnp.floor((y - PCR[1]) / VY), 0.0, CY - 1)
    czf = jnp.clip(jnp.floor((z - PCR[2]) / VZ), 0.0, CZ - 1)
    flat = ((bidx * CZ + czf.astype(jnp.int32)) * CY + cyf.astype(jnp.int32)) * CX + cxf.astype(jnp.int32)
    flat_ref[...] = flat
    ones = jnp.ones_like(x)
    vals_ref[...] = jnp.concatenate([ones, x, y, z], axis=1)


def _voxelize(points):
    return pl.pallas_call(
        _voxelize_body,
        grid=(GRID,),
        in_specs=[pl.BlockSpec((BLK, 5), lambda i: (i, 0))],
        out_specs=[
            pl.BlockSpec((BLK, 1), lambda i: (i, 0)),
            pl.BlockSpec((BLK, 4), lambda i: (i, 0)),
        ],
        out_shape=[
            jax.ShapeDtypeStruct((N, 1), jnp.int32),
            jax.ShapeDtypeStruct((N, 4), jnp.float32),
        ],
    )(points)


def _vfe1_body(pts_ref, vm_ref, w_ref, h_ref, stats_ref, acc_ref):
    @pl.when(pl.program_id(0) == 0)
    def _init():
        acc_ref[...] = jnp.zeros_like(acc_ref)

    pts = pts_ref[...]
    x = pts[:, 1:2]
    y = pts[:, 2:3]
    z = pts[:, 3:4]
    cxf = jnp.clip(jnp.floor((x - PCR[0]) / VX), 0.0, CX - 1)
    cyf = jnp.clip(jnp.floor((y - PCR[1]) / VY), 0.0, CY - 1)
    czf = jnp.clip(jnp.floor((z - PCR[2]) / VZ), 0.0, CZ - 1)
    vm = vm_ref[...]
    cnt = jnp.maximum(vm[:, 0:1], 1.0)
    vmean = vm[:, 1:4] / cnt
    xyz = pts[:, 1:4]
    f_cluster = xyz - vmean
    f_center = jnp.concatenate(
        [x - (cxf * VX + X_OFF), y - (cyf * VY + Y_OFF), z - (czf * VZ + Z_OFF)],
        axis=1,
    )
    features = jnp.concatenate([pts[:, 1:5], f_cluster, f_center], axis=1)
    h = jnp.dot(features, w_ref[...], preferred_element_type=jnp.float32)
    h_ref[...] = h
    s1 = jnp.sum(h, axis=0, keepdims=True)
    s2 = jnp.sum(h * h, axis=0, keepdims=True)
    acc_ref[...] += jnp.concatenate([s1, s2], axis=0)
    stats_ref[...] = acc_ref[...]


def _vfe1(points, vm, W1):
    return pl.pallas_call(
        _vfe1_body,
        grid=(GRID,),
        in_specs=[
            pl.BlockSpec((BLK, 5), lambda i: (i, 0)),
            pl.BlockSpec((BLK, 4), lambda i: (i, 0)),
            pl.BlockSpec((10, C1), lambda i: (0, 0)),
        ],
        out_specs=[
            pl.BlockSpec((BLK, C1), lambda i: (i, 0)),
            pl.BlockSpec((2, C1), lambda i: (0, 0)),
        ],
        out_shape=[
            jax.ShapeDtypeStruct((N, C1), jnp.float32),
            jax.ShapeDtypeStruct((2, C1), jnp.float32),
        ],
        scratch_shapes=[pltpu.VMEM((2, C1), jnp.float32)],
    )(points, vm, W1)


def _affine_relu_body(h_ref, stats_ref, g_ref, b_ref, out_ref):
    stats = stats_ref[...]
    m = stats[0:1, :] / N
    v = stats[1:2, :] / N - m * m
    scale = g_ref[...][None, :] * jax.lax.rsqrt(v + 1e-3)
    shift = b_ref[...][None, :] - m * scale
    out_ref[...] = jnp.maximum(h_ref[...] * scale + shift, 0.0)


def _affine_relu(h, stats, g, b, C):
    return pl.pallas_call(
        _affine_relu_body,
        grid=(GRID,),
        in_specs=[
            pl.BlockSpec((BLK, C), lambda i: (i, 0)),
            pl.BlockSpec((2, C), lambda i: (0, 0)),
            pl.BlockSpec((C,), lambda i: (0,)),
            pl.BlockSpec((C,), lambda i: (0,)),
        ],
        out_specs=pl.BlockSpec((BLK, C), lambda i: (i, 0)),
        out_shape=jax.ShapeDtypeStruct((N, C), jnp.float32),
    )(h, stats, g, b)


def _vfe2_body(p1_ref, v1g_ref, w_ref, h_ref, stats_ref, acc_ref):
    @pl.when(pl.program_id(0) == 0)
    def _init():
        acc_ref[...] = jnp.zeros_like(acc_ref)

    f2 = jnp.concatenate([p1_ref[...], v1g_ref[...]], axis=1)
    h = jnp.dot(f2, w_ref[...], preferred_element_type=jnp.float32)
    h_ref[...] = h
    s1 = jnp.sum(h, axis=0, keepdims=True)
    s2 = jnp.sum(h * h, axis=0, keepdims=True)
    acc_ref[...] += jnp.concatenate([s1, s2], axis=0)
    stats_ref[...] = acc_ref[...]


def _vfe2(p1, v1g, W2):
    return pl.pallas_call(
        _vfe2_body,
        grid=(GRID,),
        in_specs=[
            pl.BlockSpec((BLK, C1), lambda i: (i, 0)),
            pl.BlockSpec((BLK, C1), lambda i: (i, 0)),
            pl.BlockSpec((2 * C1, C2), lambda i: (0, 0)),
        ],
        out_specs=[
            pl.BlockSpec((BLK, C2), lambda i: (i, 0)),
            pl.BlockSpec((2, C2), lambda i: (0, 0)),
        ],
        out_shape=[
            jax.ShapeDtypeStruct((N, C2), jnp.float32),
            jax.ShapeDtypeStruct((2, C2), jnp.float32),
        ],
        scratch_shapes=[pltpu.VMEM((2, C2), jnp.float32)],
    )(p1, v1g, W2)


def kernel(points, W1, g1, b1, W2, g2, b2, batch_size):
    del batch_size
    flat2d, vals = _voxelize(points)
    flat = flat2d[:, 0]
    canvas = jax.ops.segment_sum(vals, flat, num_segments=L)
    vm = canvas[flat]
    h1, stats1 = _vfe1(points, vm, W1)
    p1 = _affine_relu(h1, stats1, g1, b1, C1)
    v1 = jnp.maximum(jax.ops.segment_max(p1, flat, num_segments=L), 0.0)
    v1g = v1[flat]
    h2, stats2 = _vfe2(p1, v1g, W2)
    p2 = _affine_relu(h2, stats2, g2, b2, C2)
    return jnp.maximum(jax.ops.segment_max(p2, flat, num_segments=L), 0.0)


# speedup vs baseline: 2.6058x; 1.1981x over previous
import functools
import jax
import jax.numpy as jnp
from jax import lax
from jax.experimental import pallas as pl
from jax.experimental.pallas import tpu as pltpu
from jax.experimental.pallas import tpu_sc as plsc

VX, VY, VZ = 0.2, 0.2, 4.0
PCR = (0.0, -40.0, -3.0, 70.4, 40.0, 1.0)
CZ, CY, CX = 1, 400, 352
B = 2
N = 262144
C1, C2 = 64, 128
L = B * CZ * CY * CX
X_OFF = VX / 2 + PCR[0]
Y_OFF = VY / 2 + PCR[1]
Z_OFF = VZ / 2 + PCR[2]

BLK = 2048
GRID = N // BLK


def _voxelize_body(pts_ref, flat_ref, vals_ref):
    pts = pts_ref[...]
    bidx = pts[:, 0:1].astype(jnp.int32)
    x = pts[:, 1:2]
    y = pts[:, 2:3]
    z = pts[:, 3:4]
    cxf = jnp.clip(jnp.floor((x - PCR[0]) / VX), 0.0, CX - 1)
    cyf = jnp.clip(jnp.floor((y - PCR[1]) / VY), 0.0, CY - 1)
    czf = jnp.clip(jnp.floor((z - PCR[2]) / VZ), 0.0, CZ - 1)
    flat = ((bidx * CZ + czf.astype(jnp.int32)) * CY + cyf.astype(jnp.int32)) * CX + cxf.astype(jnp.int32)
    flat_ref[...] = flat
    ones = jnp.ones_like(x)
    vals_ref[...] = jnp.concatenate([ones, x, y, z], axis=1)


def _voxelize(points):
    return pl.pallas_call(
        _voxelize_body,
        grid=(GRID,),
        in_specs=[pl.BlockSpec((BLK, 5), lambda i: (i, 0))],
        out_specs=[
            pl.BlockSpec((BLK, 1), lambda i: (i, 0)),
            pl.BlockSpec((BLK, 4), lambda i: (i, 0)),
        ],
        out_shape=[
            jax.ShapeDtypeStruct((N, 1), jnp.int32),
            jax.ShapeDtypeStruct((N, 4), jnp.float32),
        ],
    )(points)


_NTILES = 16
_PPT = N // _NTILES
_LPT = L // _NTILES
_GCH = 8192


def _scatter_mean_sc(flat, ones, xs, ys, zs, zeros):
    mesh = plsc.VectorSubcoreMesh(core_axis_name="c", subcore_axis_name="s", num_cores=1)
    f32 = jnp.float32

    @functools.partial(
        pl.kernel,
        mesh=mesh,
        out_type=[jax.ShapeDtypeStruct((N,), f32) for _ in range(4)],
        scratch_types=[
            pltpu.VMEM((_PPT,), jnp.int32),
            pltpu.VMEM((_PPT,), f32),
            pltpu.VMEM((_PPT,), f32),
        ]
        + [pltpu.VMEM_SHARED((L,), f32) for _ in range(4)],
    )
    def ka(flat_hbm, ones_hbm, xs_hbm, ys_hbm, zs_hbm, zeros_hbm,
           cnt_hbm, sx_hbm, sy_hbm, sz_hbm,
           idx_v, fld_v, gbuf_v, c0, c1, c2, c3):
        sid = lax.axis_index("s")
        zbase = sid * _LPT
        canvases = (c0, c1, c2, c3)
        srcs = (ones_hbm, xs_hbm, ys_hbm, zs_hbm)
        outs = (cnt_hbm, sx_hbm, sy_hbm, sz_hbm)
        rem = _LPT - _PPT
        pltpu.sync_copy(zeros_hbm.at[pl.ds(0, _PPT)], gbuf_v)
        for c in canvases:
            pltpu.sync_copy(gbuf_v, c.at[pl.ds(zbase, _PPT)])
            pltpu.sync_copy(gbuf_v.at[pl.ds(0, rem)], c.at[pl.ds(zbase + _PPT, rem)])
        plsc.subcore_barrier()
        base = sid * _PPT
        pltpu.sync_copy(flat_hbm.at[pl.ds(base, _PPT)], idx_v)
        for c, s in zip(canvases, srcs):
            pltpu.sync_copy(s.at[pl.ds(base, _PPT)], fld_v)
            pltpu.sync_copy(fld_v, c.at[idx_v], add=True)
        plsc.subcore_barrier()
        for c, o in zip(canvases, outs):
            pltpu.sync_copy(c.at[idx_v], gbuf_v)
            pltpu.sync_copy(gbuf_v, o.at[pl.ds(base, _PPT)])

    return ka(flat, ones, xs, ys, zs, zeros)


def _vfe1_body(pts_ref, vm_ref, w_ref, h_ref, stats_ref, acc_ref):
    @pl.when(pl.program_id(0) == 0)
    def _init():
        acc_ref[...] = jnp.zeros_like(acc_ref)

    pts = pts_ref[...]
    x = pts[:, 1:2]
    y = pts[:, 2:3]
    z = pts[:, 3:4]
    cxf = jnp.clip(jnp.floor((x - PCR[0]) / VX), 0.0, CX - 1)
    cyf = jnp.clip(jnp.floor((y - PCR[1]) / VY), 0.0, CY - 1)
    czf = jnp.clip(jnp.floor((z - PCR[2]) / VZ), 0.0, CZ - 1)
    vm = vm_ref[...]
    cnt = jnp.maximum(vm[:, 0:1], 1.0)
    vmean = vm[:, 1:4] / cnt
    xyz = pts[:, 1:4]
    f_cluster = xyz - vmean
    f_center = jnp.concatenate(
        [x - (cxf * VX + X_OFF), y - (cyf * VY + Y_OFF), z - (czf * VZ + Z_OFF)],
        axis=1,
    )
    features = jnp.concatenate([pts[:, 1:5], f_cluster, f_center], axis=1)
    h = jnp.dot(features, w_ref[...], preferred_element_type=jnp.float32)
    h_ref[...] = h
    s1 = jnp.sum(h, axis=0, keepdims=True)
    s2 = jnp.sum(h * h, axis=0, keepdims=True)
    acc_ref[...] += jnp.concatenate([s1, s2], axis=0)
    stats_ref[...] = acc_ref[...]


def _vfe1(points, vm, W1):
    return pl.pallas_call(
        _vfe1_body,
        grid=(GRID,),
        in_specs=[
            pl.BlockSpec((BLK, 5), lambda i: (i, 0)),
            pl.BlockSpec((BLK, 4), lambda i: (i, 0)),
            pl.BlockSpec((10, C1), lambda i: (0, 0)),
        ],
        out_specs=[
            pl.BlockSpec((BLK, C1), lambda i: (i, 0)),
            pl.BlockSpec((2, C1), lambda i: (0, 0)),
        ],
        out_shape=[
            jax.ShapeDtypeStruct((N, C1), jnp.float32),
            jax.ShapeDtypeStruct((2, C1), jnp.float32),
        ],
        scratch_shapes=[pltpu.VMEM((2, C1), jnp.float32)],
    )(points, vm, W1)


def _affine_relu_body(h_ref, stats_ref, g_ref, b_ref, out_ref):
    stats = stats_ref[...]
    m = stats[0:1, :] / N
    v = stats[1:2, :] / N - m * m
    scale = g_ref[...][None, :] * jax.lax.rsqrt(v + 1e-3)
    shift = b_ref[...][None, :] - m * scale
    out_ref[...] = jnp.maximum(h_ref[...] * scale + shift, 0.0)


def _affine_relu(h, stats, g, b, C):
    return pl.pallas_call(
        _affine_relu_body,
        grid=(GRID,),
        in_specs=[
            pl.BlockSpec((BLK, C), lambda i: (i, 0)),
            pl.BlockSpec((2, C), lambda i: (0, 0)),
            pl.BlockSpec((C,), lambda i: (0,)),
            pl.BlockSpec((C,), lambda i: (0,)),
        ],
        out_specs=pl.BlockSpec((BLK, C), lambda i: (i, 0)),
        out_shape=jax.ShapeDtypeStruct((N, C), jnp.float32),
    )(h, stats, g, b)


def _vfe2_body(p1_ref, v1g_ref, w_ref, h_ref, stats_ref, acc_ref):
    @pl.when(pl.program_id(0) == 0)
    def _init():
        acc_ref[...] = jnp.zeros_like(acc_ref)

    f2 = jnp.concatenate([p1_ref[...], v1g_ref[...]], axis=1)
    h = jnp.dot(f2, w_ref[...], preferred_element_type=jnp.float32)
    h_ref[...] = h
    s1 = jnp.sum(h, axis=0, keepdims=True)
    s2 = jnp.sum(h * h, axis=0, keepdims=True)
    acc_ref[...] += jnp.concatenate([s1, s2], axis=0)
    stats_ref[...] = acc_ref[...]


def _vfe2(p1, v1g, W2):
    return pl.pallas_call(
        _vfe2_body,
        grid=(GRID,),
        in_specs=[
            pl.BlockSpec((BLK, C1), lambda i: (i, 0)),
            pl.BlockSpec((BLK, C1), lambda i: (i, 0)),
            pl.BlockSpec((2 * C1, C2), lambda i: (0, 0)),
        ],
        out_specs=[
            pl.BlockSpec((BLK, C2), lambda i: (i, 0)),
            pl.BlockSpec((2, C2), lambda i: (0, 0)),
        ],
        out_shape=[
            jax.ShapeDtypeStruct((N, C2), jnp.float32),
            jax.ShapeDtypeStruct((2, C2), jnp.float32),
        ],
        scratch_shapes=[pltpu.VMEM((2, C2), jnp.float32)],
    )(p1, v1g, W2)


def kernel(points, W1, g1, b1, W2, g2, b2, batch_size):
    del batch_size
    flat2d, vals = _voxelize(points)
    del vals
    flat = flat2d[:, 0]
    cnt_g, sx_g, sy_g, sz_g = _scatter_mean_sc(
        flat, jnp.ones((N,), jnp.float32), points[:, 1], points[:, 2], points[:, 3],
        jnp.zeros((L,), jnp.float32))
    vm = jnp.stack([cnt_g, sx_g, sy_g, sz_g], axis=1)
    h1, stats1 = _vfe1(points, vm, W1)
    p1 = _affine_relu(h1, stats1, g1, b1, C1)
    v1 = jnp.maximum(jax.ops.segment_max(p1, flat, num_segments=L), 0.0)
    v1g = v1[flat]
    h2, stats2 = _vfe2(p1, v1g, W2)
    p2 = _affine_relu(h2, stats2, g2, b2, C2)
    return jnp.maximum(jax.ops.segment_max(p2, flat, num_segments=L), 0.0)
